# Initial kernel scaffold; baseline (speedup 1.0000x reference)
import jax
import jax.numpy as jnp
from jax.experimental import pallas as pl


def kernel(x_drug, x_protein, edge_index_dp, edge_index_pd, W1_dp_l, W1_dp_r, b1_dp, W1_pd_l, W1_pd_r, b1_pd, W2_dp_l, W2_dp_r, b2_dp, W2_pd_l, W2_pd_r, b2_pd):
    raise NotImplementedError("write your pallas kernel here")


# speedup vs baseline: 1.2593x; 1.2593x over previous
import functools

import jax
import jax.numpy as jnp
from jax.experimental import pallas as pl
from jax.experimental.pallas import tpu as pltpu
from jax.experimental.pallas import tpu_sc as plsc

N_DRUG = 10000
N_PROT = 50000
D = 128

E = 300000
NW = 32
BLK = 128
EPAD = 303104
TILE_E = EPAD // NW
NBLK = TILE_E // BLK

P_CHUNK = 8704
P_PASSES = 6
P_PAD = P_CHUNK * P_PASSES
D_CHUNK = 5120
D_PASSES = 2
D_PAD = D_CHUNK * D_PASSES


def _copy_rows(src, dst, r0, nrows, unit=128):
  full, rem = divmod(nrows, unit)
  for k in range(full):
    pltpu.sync_copy(src.at[pl.ds(r0 + k * unit, unit)],
                    dst.at[pl.ds(r0 + k * unit, unit)])
  if rem:
    pltpu.sync_copy(src.at[pl.ds(r0 + full * unit, rem)],
                    dst.at[pl.ds(r0 + full * unit, rem)])


def _make_sc_aggr(n_pad, chunk, n_passes, gather=True):
  mesh = plsc.VectorSubcoreMesh(core_axis_name="c", subcore_axis_name="s")
  acc_rows = chunk + BLK
  zrows = acc_rows // 16
  srows = chunk // 16

  scratch = [
      pltpu.VMEM((TILE_E,), jnp.int32),
      pltpu.VMEM((TILE_E,), jnp.int32),
      pltpu.VMEM((BLK, D), jnp.float32),
      pltpu.VMEM((BLK,), jnp.int32),
      pltpu.VMEM((BLK, D), jnp.float32),
      pltpu.VMEM_SHARED((acc_rows, D), jnp.float32),
  ]

  @functools.partial(pl.kernel,
                     out_type=jax.ShapeDtypeStruct((2, n_pad, D), jnp.float32),
                     mesh=mesh, scratch_types=scratch)
  def aggr(x_hbm, src_hbm, dst_hbm, zeros_hbm, s_out,
           src_v, dst_v, rows_v, loc_v, zer_v, acc):
    c = jax.lax.axis_index("c")
    s = jax.lax.axis_index("s")
    w = c * 16 + s
    base = w * TILE_E

    if gather:
      pltpu.sync_copy(src_hbm.at[pl.ds(base, TILE_E)], src_v)
    else:
      pltpu.sync_copy(x_hbm, rows_v)
    pltpu.sync_copy(dst_hbm.at[pl.ds(base, TILE_E)], dst_v)
    pltpu.sync_copy(zeros_hbm, zer_v)

    for p in range(n_passes):
      lo = p * chunk
      hi = lo + chunk

      full, rem = divmod(zrows, BLK)
      for k in range(full):
        pltpu.sync_copy(zer_v, acc.at[pl.ds(s * zrows + k * BLK, BLK)])
      if rem:
        pltpu.sync_copy(zer_v.at[pl.ds(0, rem)],
                        acc.at[pl.ds(s * zrows + full * BLK, rem)])
      plsc.subcore_barrier()

      @pl.loop(0, NBLK)
      def _(b):
        e0 = b * BLK
        for j in range(BLK // 16):
          d16 = dst_v[pl.ds(e0 + j * 16, 16)]
          m = (d16 >= lo) & (d16 < hi)
          loc_v[pl.ds(j * 16, 16)] = jnp.where(m, d16 - lo, chunk)
        if gather:
          pltpu.sync_copy(x_hbm.at[src_v.at[pl.ds(e0, BLK)]], rows_v)
        pltpu.sync_copy(rows_v, acc.at[loc_v], add=True)

      plsc.subcore_barrier()

      full, rem = divmod(srows, BLK)
      for k in range(full):
        r0 = s * srows + k * BLK
        pltpu.sync_copy(acc.at[pl.ds(r0, BLK)],
                        s_out.at[c, pl.ds(lo + r0, BLK)])
      if rem:
        r0 = s * srows + full * BLK
        pltpu.sync_copy(acc.at[pl.ds(r0, rem)],
                        s_out.at[c, pl.ds(lo + r0, rem)])
      plsc.subcore_barrier()

  return aggr


def _tc_layer(s0, s1, c0, c1, x, w_l, w_r, b, relu, norm, blk=512):
  n = s0.shape[0]

  def body(s0_r, s1_r, c0_r, c1_r, x_r, wl_r, wr_r, b_r, o_r):
    cnt = c0_r[:, 0:1] + c1_r[:, 0:1]
    a = (s0_r[...] + s1_r[...]) / jnp.maximum(cnt, 1.0)
    h = (jnp.dot(a, wl_r[...], preferred_element_type=jnp.float32)
         + jnp.dot(x_r[...], wr_r[...], preferred_element_type=jnp.float32)
         + b_r[...])
    if relu:
      h = jnp.maximum(h, 0.0)
    if norm:
      nrm = jnp.sqrt(jnp.sum(h * h, axis=-1, keepdims=True))
      h = h / jnp.maximum(nrm, 1e-12)
    o_r[...] = h

  return pl.pallas_call(
      body,
      grid=(n // blk,),
      in_specs=[
          pl.BlockSpec((blk, D), lambda i: (i, 0)),
          pl.BlockSpec((blk, D), lambda i: (i, 0)),
          pl.BlockSpec((blk, D), lambda i: (i, 0)),
          pl.BlockSpec((blk, D), lambda i: (i, 0)),
          pl.BlockSpec((blk, D), lambda i: (i, 0)),
          pl.BlockSpec((D, D), lambda i: (0, 0)),
          pl.BlockSpec((D, D), lambda i: (0, 0)),
          pl.BlockSpec((1, D), lambda i: (0, 0)),
      ],
      out_specs=pl.BlockSpec((blk, D), lambda i: (i, 0)),
      out_shape=jax.ShapeDtypeStruct((n, D), jnp.float32),
  )(s0, s1, c0, c1, x, w_l, w_r, b)


def kernel(x_drug, x_protein, edge_index_dp, edge_index_pd,
           W1_dp_l, W1_dp_r, b1_dp, W1_pd_l, W1_pd_r, b1_pd,
           W2_dp_l, W2_dp_r, b2_dp, W2_pd_l, W2_pd_r, b2_pd):
  f32 = jnp.float32
  i32 = jnp.int32

  def pad_edges(ei, dst_pad_val):
    src = jnp.pad(ei[0].astype(i32), (0, EPAD - E))
    dst = jnp.pad(ei[1].astype(i32), (0, EPAD - E),
                  constant_values=dst_pad_val)
    return src, dst

  src_dp, dst_dp = pad_edges(edge_index_dp, P_PAD)
  src_pd, dst_pd = pad_edges(edge_index_pd, D_PAD)

  x_p = jnp.pad(x_protein.astype(f32), ((0, P_PAD - N_PROT), (0, 0)))
  x_d = jnp.pad(x_drug.astype(f32), ((0, D_PAD - N_DRUG), (0, 0)))

  ones = jnp.ones((BLK, D), f32)
  zer = jnp.zeros((BLK, D), f32)

  aggr_p = _make_sc_aggr(P_PAD, P_CHUNK, P_PASSES)
  aggr_d = _make_sc_aggr(D_PAD, D_CHUNK, D_PASSES)
  count_p = _make_sc_aggr(P_PAD, P_CHUNK, P_PASSES, gather=False)
  count_d = _make_sc_aggr(D_PAD, D_CHUNK, D_PASSES, gather=False)

  c_p = count_p(ones, src_dp, dst_dp, zer)
  c_d = count_d(ones, src_pd, dst_pd, zer)

  s1p = aggr_p(x_drug, src_dp, dst_dp, zer)
  s1d = aggr_d(x_protein, src_pd, dst_pd, zer)
  h_p = _tc_layer(s1p[0], s1p[1], c_p[0], c_p[1], x_p,
                  W1_dp_l, W1_dp_r, b1_dp.reshape(1, D), True, False)
  h_d = _tc_layer(s1d[0], s1d[1], c_d[0], c_d[1], x_d,
                  W1_pd_l, W1_pd_r, b1_pd.reshape(1, D), True, False)

  s2p = aggr_p(h_d, src_dp, dst_dp, zer)
  s2d = aggr_d(h_p, src_pd, dst_pd, zer)
  z_p = _tc_layer(s2p[0], s2p[1], c_p[0], c_p[1], h_p,
                  W2_dp_l, W2_dp_r, b2_dp.reshape(1, D), False, True)
  z_d = _tc_layer(s2d[0], s2d[1], c_d[0], c_d[1], h_d,
                  W2_pd_l, W2_pd_r, b2_pd.reshape(1, D), False, True)

  return (z_d[:N_DRUG], z_p[:N_PROT])


# speedup vs baseline: 2.9451x; 2.3387x over previous
import dataclasses
import functools

import jax
import jax.numpy as jnp
from jax.experimental import pallas as pl
from jax.experimental.pallas import tpu as pltpu
from jax.experimental.pallas import tpu_sc as plsc

N_DRUG = 10000
N_PROT = 50000
D = 128

E = 300000
NW = 32
BLK = 128
EPAD = 303104
TILE_E = EPAD // NW
NBLK = TILE_E // BLK

P_CHUNK = 8704
P_PASSES = 6
P_PAD = P_CHUNK * P_PASSES
D_CHUNK = 5120
D_PASSES = 2
D_PAD = D_CHUNK * D_PASSES


def _copy_rows(src, dst, r0, nrows, unit=128):
  full, rem = divmod(nrows, unit)
  for k in range(full):
    pltpu.sync_copy(src.at[pl.ds(r0 + k * unit, unit)],
                    dst.at[pl.ds(r0 + k * unit, unit)])
  if rem:
    pltpu.sync_copy(src.at[pl.ds(r0 + full * unit, rem)],
                    dst.at[pl.ds(r0 + full * unit, rem)])


def _make_sc_aggr(n_pad, chunk, n_passes, gather=True):
  mesh = plsc.VectorSubcoreMesh(core_axis_name="c", subcore_axis_name="s")
  acc_rows = chunk + BLK
  zrows = acc_rows // 16
  srows = chunk // 16
  NV = TILE_E // 16
  NBC = TILE_E // BLK + 1

  scratch = [
      pltpu.VMEM((TILE_E,), jnp.int32),
      pltpu.VMEM((TILE_E,), jnp.int32),
      pltpu.VMEM((NBC, BLK), jnp.int32),
      pltpu.VMEM((NBC, BLK), jnp.int32),
      pltpu.VMEM((BLK, D), jnp.float32),
      pltpu.VMEM_SHARED((acc_rows, D), jnp.float32),
      pltpu.VMEM((16,), jnp.int32),
  ]

  cp = pltpu.CompilerParams()
  if "needs_layout_passes" in pltpu.CompilerParams.__dataclass_fields__:
    cp = dataclasses.replace(cp, needs_layout_passes=False)

  @functools.partial(pl.kernel,
                     out_type=jax.ShapeDtypeStruct((2, n_pad, D), jnp.float32),
                     mesh=mesh, scratch_types=scratch, compiler_params=cp)
  def aggr(x_hbm, src_hbm, dst_hbm, zeros_hbm, s_out,
           src_v, dst_v, csrc, cloc, rows_v, acc, loc_v16):
    c = jax.lax.axis_index("c")
    s = jax.lax.axis_index("s")
    w = c * 16 + s
    base = w * TILE_E

    if gather:
      pltpu.sync_copy(src_hbm.at[pl.ds(base, TILE_E)], src_v)
    pltpu.sync_copy(dst_hbm.at[pl.ds(base, TILE_E)], dst_v)
    lanes = jax.lax.iota(jnp.int32, 16)

    for p in range(n_passes):
      lo = p * chunk
      hi = lo + chunk

      pltpu.sync_copy(zeros_hbm, rows_v)
      full, rem = divmod(zrows, BLK)
      for k in range(full):
        pltpu.sync_copy(rows_v, acc.at[pl.ds(s * zrows + k * BLK, BLK)])
      if rem:
        pltpu.sync_copy(rows_v.at[pl.ds(0, rem)],
                        acc.at[pl.ds(s * zrows + full * BLK, rem)])
      if not gather:
        pltpu.sync_copy(x_hbm, rows_v)

      loc_v16[...] = jnp.zeros((16,), jnp.int32)

      @pl.loop(0, NV)
      def _(v):
        d16 = dst_v[pl.ds(v * 16, 16)]
        m = (d16 >= lo) & (d16 < hi)
        pc = plsc.all_reduce_population_count(m)
        off16 = loc_v16[...]
        pos = off16 + plsc.cumsum(m.astype(jnp.int32)) - 1
        row = jax.lax.shift_right_logical(pos, 7)
        col = jax.lax.bitwise_and(pos, BLK - 1)
        if gather:
          s16 = src_v[pl.ds(v * 16, 16)]
          plsc.store_scatter(csrc, [row, col], s16, mask=m)
        plsc.store_scatter(cloc, [row, col], d16 - lo, mask=m)
        loc_v16[...] = off16 + pc

      off = jnp.sum(jnp.where(jax.lax.iota(jnp.int32, 16) == 0,
                              loc_v16[...], 0))

      for k in range(BLK // 16):
        pos = off + k * 16 + lanes
        row = jax.lax.shift_right_logical(pos, 7)
        col = jax.lax.bitwise_and(pos, BLK - 1)
        if gather:
          plsc.store_scatter(csrc, [row, col], jnp.zeros((16,), jnp.int32))
        plsc.store_scatter(cloc, [row, col],
                           jnp.full((16,), chunk, jnp.int32))
      nb = (off + BLK - 1) // BLK
      plsc.subcore_barrier()

      @pl.loop(0, NBLK)
      def _(b):
        @pl.when(b < nb)
        def _():
          if gather:
            pltpu.sync_copy(x_hbm.at[csrc.at[b]], rows_v)
          pltpu.sync_copy(rows_v, acc.at[cloc.at[b]], add=True)

      plsc.subcore_barrier()

      full, rem = divmod(srows, BLK)
      for k in range(full):
        r0 = s * srows + k * BLK
        pltpu.sync_copy(acc.at[pl.ds(r0, BLK)],
                        s_out.at[c, pl.ds(lo + r0, BLK)])
      if rem:
        r0 = s * srows + full * BLK
        pltpu.sync_copy(acc.at[pl.ds(r0, rem)],
                        s_out.at[c, pl.ds(lo + r0, rem)])
      plsc.subcore_barrier()

  return aggr


def _tc_layer(s0, s1, c0, c1, x, w_l, w_r, b, relu, norm, blk=512):
  n = s0.shape[0]

  def body(s0_r, s1_r, c0_r, c1_r, x_r, wl_r, wr_r, b_r, o_r):
    cnt = c0_r[:, 0:1] + c1_r[:, 0:1]
    a = (s0_r[...] + s1_r[...]) / jnp.maximum(cnt, 1.0)
    h = (jnp.dot(a, wl_r[...], preferred_element_type=jnp.float32)
         + jnp.dot(x_r[...], wr_r[...], preferred_element_type=jnp.float32)
         + b_r[...])
    if relu:
      h = jnp.maximum(h, 0.0)
    if norm:
      nrm = jnp.sqrt(jnp.sum(h * h, axis=-1, keepdims=True))
      h = h / jnp.maximum(nrm, 1e-12)
    o_r[...] = h

  return pl.pallas_call(
      body,
      grid=(n // blk,),
      in_specs=[
          pl.BlockSpec((blk, D), lambda i: (i, 0)),
          pl.BlockSpec((blk, D), lambda i: (i, 0)),
          pl.BlockSpec((blk, D), lambda i: (i, 0)),
          pl.BlockSpec((blk, D), lambda i: (i, 0)),
          pl.BlockSpec((blk, D), lambda i: (i, 0)),
          pl.BlockSpec((D, D), lambda i: (0, 0)),
          pl.BlockSpec((D, D), lambda i: (0, 0)),
          pl.BlockSpec((1, D), lambda i: (0, 0)),
      ],
      out_specs=pl.BlockSpec((blk, D), lambda i: (i, 0)),
      out_shape=jax.ShapeDtypeStruct((n, D), jnp.float32),
  )(s0, s1, c0, c1, x, w_l, w_r, b)


def kernel(x_drug, x_protein, edge_index_dp, edge_index_pd,
           W1_dp_l, W1_dp_r, b1_dp, W1_pd_l, W1_pd_r, b1_pd,
           W2_dp_l, W2_dp_r, b2_dp, W2_pd_l, W2_pd_r, b2_pd):
  f32 = jnp.float32
  i32 = jnp.int32

  def pad_edges(ei, dst_pad_val):
    src = jnp.pad(ei[0].astype(i32), (0, EPAD - E))
    dst = jnp.pad(ei[1].astype(i32), (0, EPAD - E),
                  constant_values=dst_pad_val)
    return src, dst

  src_dp, dst_dp = pad_edges(edge_index_dp, P_PAD)
  src_pd, dst_pd = pad_edges(edge_index_pd, D_PAD)

  x_p = jnp.pad(x_protein.astype(f32), ((0, P_PAD - N_PROT), (0, 0)))
  x_d = jnp.pad(x_drug.astype(f32), ((0, D_PAD - N_DRUG), (0, 0)))

  ones = jnp.ones((BLK, D), f32)
  zer = jnp.zeros((BLK, D), f32)

  aggr_p = _make_sc_aggr(P_PAD, P_CHUNK, P_PASSES)
  aggr_d = _make_sc_aggr(D_PAD, D_CHUNK, D_PASSES)
  count_p = _make_sc_aggr(P_PAD, P_CHUNK, P_PASSES, gather=False)
  count_d = _make_sc_aggr(D_PAD, D_CHUNK, D_PASSES, gather=False)

  c_p = count_p(ones, src_dp, dst_dp, zer)
  c_d = count_d(ones, src_pd, dst_pd, zer)

  s1p = aggr_p(x_drug, src_dp, dst_dp, zer)
  s1d = aggr_d(x_protein, src_pd, dst_pd, zer)
  h_p = _tc_layer(s1p[0], s1p[1], c_p[0], c_p[1], x_p,
                  W1_dp_l, W1_dp_r, b1_dp.reshape(1, D), True, False)
  h_d = _tc_layer(s1d[0], s1d[1], c_d[0], c_d[1], x_d,
                  W1_pd_l, W1_pd_r, b1_pd.reshape(1, D), True, False)

  s2p = aggr_p(h_d, src_dp, dst_dp, zer)
  s2d = aggr_d(h_p, src_pd, dst_pd, zer)
  z_p = _tc_layer(s2p[0], s2p[1], c_p[0], c_p[1], h_p,
                  W2_dp_l, W2_dp_r, b2_dp.reshape(1, D), False, True)
  z_d = _tc_layer(s2d[0], s2d[1], c_d[0], c_d[1], h_d,
                  W2_pd_l, W2_pd_r, b2_pd.reshape(1, D), False, True)

  return (z_d[:N_DRUG], z_p[:N_PROT])


# speedup vs baseline: 3.0135x; 1.0232x over previous
import dataclasses
import functools

import jax
import jax.numpy as jnp
from jax.experimental import pallas as pl
from jax.experimental.pallas import tpu as pltpu
from jax.experimental.pallas import tpu_sc as plsc

N_DRUG = 10000
N_PROT = 50000
D = 128

E = 300000
NW = 32
BLK = 128
EPAD = 303104
TILE_E = EPAD // NW
NBLK = TILE_E // BLK

P_CHUNK = 7168
P_PASSES = 7
P_PAD = P_CHUNK * P_PASSES
D_CHUNK = 5120
D_PASSES = 2
D_PAD = D_CHUNK * D_PASSES


def _copy_rows(src, dst, r0, nrows, unit=128):
  full, rem = divmod(nrows, unit)
  for k in range(full):
    pltpu.sync_copy(src.at[pl.ds(r0 + k * unit, unit)],
                    dst.at[pl.ds(r0 + k * unit, unit)])
  if rem:
    pltpu.sync_copy(src.at[pl.ds(r0 + full * unit, rem)],
                    dst.at[pl.ds(r0 + full * unit, rem)])


def _make_sc_aggr(n_pad, chunk, n_passes, gather=True):
  mesh = plsc.VectorSubcoreMesh(core_axis_name="c", subcore_axis_name="s")
  acc_rows = chunk + BLK
  zrows = acc_rows // 16
  srows = chunk // 16
  NV = TILE_E // 16
  NBC = TILE_E // BLK + 1

  scratch = [
      pltpu.VMEM((TILE_E,), jnp.int32),
      pltpu.VMEM((TILE_E,), jnp.int32),
      pltpu.VMEM((NBC, BLK), jnp.int32),
      pltpu.VMEM((NBC, BLK), jnp.int32),
      pltpu.VMEM((BLK, D), jnp.float32),
      pltpu.VMEM((BLK, D), jnp.float32) if gather else None,
      pltpu.VMEM_SHARED((acc_rows, D), jnp.float32),
      pltpu.VMEM((16,), jnp.int32),
      pltpu.SemaphoreType.DMA,
      pltpu.SemaphoreType.DMA,
      pltpu.SemaphoreType.DMA,
      pltpu.SemaphoreType.DMA,
      pltpu.SemaphoreType.DMA,
  ]
  scratch = [t for t in scratch if t is not None]

  cp = pltpu.CompilerParams()
  if "needs_layout_passes" in pltpu.CompilerParams.__dataclass_fields__:
    cp = dataclasses.replace(cp, needs_layout_passes=False)

  @functools.partial(pl.kernel,
                     out_type=jax.ShapeDtypeStruct((2, n_pad, D), jnp.float32),
                     mesh=mesh, scratch_types=scratch, compiler_params=cp)
  def aggr(x_hbm, src_hbm, dst_hbm, zeros_hbm, s_out, *rest):
    if gather:
      (src_v, dst_v, csrc, cloc, rows0, rows1, acc, loc_v16,
       sg0, sg1, ss0, ss1, sw) = rest
    else:
      (src_v, dst_v, csrc, cloc, rows0, acc, loc_v16,
       sg0, sg1, ss0, ss1, sw) = rest
      rows1 = rows0
    c = jax.lax.axis_index("c")
    s = jax.lax.axis_index("s")
    w = c * 16 + s
    base = w * TILE_E

    if gather:
      pltpu.sync_copy(src_hbm.at[pl.ds(base, TILE_E)], src_v)
    pltpu.sync_copy(dst_hbm.at[pl.ds(base, TILE_E)], dst_v)
    lanes = jax.lax.iota(jnp.int32, 16)

    def g_start(b, buf, sem):
      pltpu.async_copy(x_hbm.at[csrc.at[b]], buf, sem)

    def g_wait(b, buf, sem):
      pltpu.make_async_copy(x_hbm.at[csrc.at[b]], buf, sem).wait()

    def s_start(b, buf, sem):
      pltpu.async_copy(buf, acc.at[cloc.at[b]], sem, add=True)

    def s_wait(b, buf, sem):
      pltpu.make_async_copy(buf, acc.at[cloc.at[b]], sem).wait()

    for p in range(n_passes):
      lo = p * chunk
      hi = lo + chunk

      pltpu.sync_copy(zeros_hbm, rows0)
      full, rem = divmod(zrows, BLK)
      for k in range(full):
        pltpu.async_copy(rows0, acc.at[pl.ds(s * zrows + k * BLK, BLK)], sw)
      if rem:
        pltpu.async_copy(rows0.at[pl.ds(0, rem)],
                         acc.at[pl.ds(s * zrows + full * BLK, rem)], sw)

      loc_v16[...] = jnp.zeros((16,), jnp.int32)

      @pl.loop(0, NV)
      def _(v):
        d16 = dst_v[pl.ds(v * 16, 16)]
        m = (d16 >= lo) & (d16 < hi)
        pc = plsc.all_reduce_population_count(m)
        off16 = loc_v16[...]
        pos = off16 + plsc.cumsum(m.astype(jnp.int32)) - 1
        row = jax.lax.shift_right_logical(pos, 7)
        col = jax.lax.bitwise_and(pos, BLK - 1)
        if gather:
          s16 = src_v[pl.ds(v * 16, 16)]
          plsc.store_scatter(csrc, [row, col], s16, mask=m)
        plsc.store_scatter(cloc, [row, col], d16 - lo, mask=m)
        loc_v16[...] = off16 + pc

      off = jnp.sum(jnp.where(jax.lax.iota(jnp.int32, 16) == 0,
                              loc_v16[...], 0))

      for k in range(BLK // 16):
        pos = off + k * 16 + lanes
        row = jax.lax.shift_right_logical(pos, 7)
        col = jax.lax.bitwise_and(pos, BLK - 1)
        if gather:
          plsc.store_scatter(csrc, [row, col], jnp.zeros((16,), jnp.int32))
        plsc.store_scatter(cloc, [row, col],
                           jnp.full((16,), chunk, jnp.int32))
      nb = (off + BLK - 1) // BLK

      for k in range(full):
        pltpu.make_async_copy(rows0,
                              acc.at[pl.ds(s * zrows + k * BLK, BLK)],
                              sw).wait()
      if rem:
        pltpu.make_async_copy(rows0.at[pl.ds(0, rem)],
                              acc.at[pl.ds(s * zrows + full * BLK, rem)],
                              sw).wait()
      if not gather:
        pltpu.sync_copy(x_hbm, rows0)
      plsc.subcore_barrier()

      if gather:
        @pl.when(nb > 0)
        def _():
          g_start(0, rows0, sg0)

        @pl.loop(0, NBLK)
        def _(b):
          @pl.when(b < nb)
          def _():
            even = jax.lax.bitwise_and(b, 1) == 0
            odd = jnp.logical_not(even)

            @pl.when((b >= 1) & even)
            def _():
              s_wait(b - 1, rows1, ss1)

            @pl.when((b >= 1) & odd)
            def _():
              s_wait(b - 1, rows0, ss0)

            @pl.when((b + 1 < nb) & even)
            def _():
              g_start(b + 1, rows1, sg1)

            @pl.when((b + 1 < nb) & odd)
            def _():
              g_start(b + 1, rows0, sg0)

            @pl.when(even)
            def _():
              g_wait(b, rows0, sg0)
              s_start(b, rows0, ss0)

            @pl.when(odd)
            def _():
              g_wait(b, rows1, sg1)
              s_start(b, rows1, ss1)

        @pl.when(nb > 0)
        def _():
          last = nb - 1
          le = jax.lax.bitwise_and(last, 1) == 0

          @pl.when(le)
          def _():
            s_wait(last, rows0, ss0)

          @pl.when(jnp.logical_not(le))
          def _():
            s_wait(last, rows1, ss1)
      else:
        @pl.loop(0, NBLK)
        def _(b):
          @pl.when(b < nb)
          def _():
            @pl.when(b >= 1)
            def _():
              s_wait(b - 1, rows0, ss0)
            s_start(b, rows0, ss0)

        @pl.when(nb > 0)
        def _():
          s_wait(nb - 1, rows0, ss0)

      plsc.subcore_barrier()

      full, rem = divmod(srows, BLK)
      for k in range(full):
        r0 = s * srows + k * BLK
        pltpu.async_copy(acc.at[pl.ds(r0, BLK)],
                         s_out.at[c, pl.ds(lo + r0, BLK)], sw)
      if rem:
        r0 = s * srows + full * BLK
        pltpu.async_copy(acc.at[pl.ds(r0, rem)],
                         s_out.at[c, pl.ds(lo + r0, rem)], sw)
      for k in range(full):
        r0 = s * srows + k * BLK
        pltpu.make_async_copy(acc.at[pl.ds(r0, BLK)],
                              s_out.at[c, pl.ds(lo + r0, BLK)], sw).wait()
      if rem:
        r0 = s * srows + full * BLK
        pltpu.make_async_copy(acc.at[pl.ds(r0, rem)],
                              s_out.at[c, pl.ds(lo + r0, rem)], sw).wait()
      plsc.subcore_barrier()

  return aggr


def _tc_layer(s0, s1, c0, c1, x, w_l, w_r, b, relu, norm, blk=512):
  n = s0.shape[0]

  def body(s0_r, s1_r, c0_r, c1_r, x_r, wl_r, wr_r, b_r, o_r):
    cnt = c0_r[:, 0:1] + c1_r[:, 0:1]
    a = (s0_r[...] + s1_r[...]) / jnp.maximum(cnt, 1.0)
    h = (jnp.dot(a, wl_r[...], preferred_element_type=jnp.float32)
         + jnp.dot(x_r[...], wr_r[...], preferred_element_type=jnp.float32)
         + b_r[...])
    if relu:
      h = jnp.maximum(h, 0.0)
    if norm:
      nrm = jnp.sqrt(jnp.sum(h * h, axis=-1, keepdims=True))
      h = h / jnp.maximum(nrm, 1e-12)
    o_r[...] = h

  return pl.pallas_call(
      body,
      grid=(n // blk,),
      in_specs=[
          pl.BlockSpec((blk, D), lambda i: (i, 0)),
          pl.BlockSpec((blk, D), lambda i: (i, 0)),
          pl.BlockSpec((blk, D), lambda i: (i, 0)),
          pl.BlockSpec((blk, D), lambda i: (i, 0)),
          pl.BlockSpec((blk, D), lambda i: (i, 0)),
          pl.BlockSpec((D, D), lambda i: (0, 0)),
          pl.BlockSpec((D, D), lambda i: (0, 0)),
          pl.BlockSpec((1, D), lambda i: (0, 0)),
      ],
      out_specs=pl.BlockSpec((blk, D), lambda i: (i, 0)),
      out_shape=jax.ShapeDtypeStruct((n, D), jnp.float32),
  )(s0, s1, c0, c1, x, w_l, w_r, b)


def kernel(x_drug, x_protein, edge_index_dp, edge_index_pd,
           W1_dp_l, W1_dp_r, b1_dp, W1_pd_l, W1_pd_r, b1_pd,
           W2_dp_l, W2_dp_r, b2_dp, W2_pd_l, W2_pd_r, b2_pd):
  f32 = jnp.float32
  i32 = jnp.int32

  def pad_edges(ei, dst_pad_val):
    src = jnp.pad(ei[0].astype(i32), (0, EPAD - E))
    dst = jnp.pad(ei[1].astype(i32), (0, EPAD - E),
                  constant_values=dst_pad_val)
    return src, dst

  src_dp, dst_dp = pad_edges(edge_index_dp, P_PAD)
  src_pd, dst_pd = pad_edges(edge_index_pd, D_PAD)

  x_p = jnp.pad(x_protein.astype(f32), ((0, P_PAD - N_PROT), (0, 0)))
  x_d = jnp.pad(x_drug.astype(f32), ((0, D_PAD - N_DRUG), (0, 0)))

  ones = jnp.ones((BLK, D), f32)
  zer = jnp.zeros((BLK, D), f32)

  aggr_p = _make_sc_aggr(P_PAD, P_CHUNK, P_PASSES)
  aggr_d = _make_sc_aggr(D_PAD, D_CHUNK, D_PASSES)
  count_p = _make_sc_aggr(P_PAD, P_CHUNK, P_PASSES, gather=False)
  count_d = _make_sc_aggr(D_PAD, D_CHUNK, D_PASSES, gather=False)

  c_p = count_p(ones, src_dp, dst_dp, zer)
  c_d = count_d(ones, src_pd, dst_pd, zer)

  s1p = aggr_p(x_drug, src_dp, dst_dp, zer)
  s1d = aggr_d(x_protein, src_pd, dst_pd, zer)
  h_p = _tc_layer(s1p[0], s1p[1], c_p[0], c_p[1], x_p,
                  W1_dp_l, W1_dp_r, b1_dp.reshape(1, D), True, False)
  h_d = _tc_layer(s1d[0], s1d[1], c_d[0], c_d[1], x_d,
                  W1_pd_l, W1_pd_r, b1_pd.reshape(1, D), True, False)

  s2p = aggr_p(h_d, src_dp, dst_dp, zer)
  s2d = aggr_d(h_p, src_pd, dst_pd, zer)
  z_p = _tc_layer(s2p[0], s2p[1], c_p[0], c_p[1], h_p,
                  W2_dp_l, W2_dp_r, b2_dp.reshape(1, D), False, True)
  z_d = _tc_layer(s2d[0], s2d[1], c_d[0], c_d[1], h_d,
                  W2_pd_l, W2_pd_r, b2_pd.reshape(1, D), False, True)

  return (z_d[:N_DRUG], z_p[:N_PROT])


# speedup vs baseline: 3.3720x; 1.1190x over previous
import dataclasses
import functools

import jax
import jax.numpy as jnp
from jax.experimental import pallas as pl
from jax.experimental.pallas import tpu as pltpu
from jax.experimental.pallas import tpu_sc as plsc

N_DRUG = 10000
N_PROT = 50000
D = 128

E = 300000
NW = 32
BLK = 128
EPAD = 303104
TILE_E = EPAD // NW
NBLK = TILE_E // BLK

P_CHUNK = 8704
P_PASSES = 6
P_PAD = P_CHUNK * P_PASSES
P_CAPB = NBLK + P_PASSES + 1
D_CHUNK = 5120
D_PASSES = 2
D_PAD = D_CHUNK * D_PASSES
D_CAPB = NBLK + D_PASSES + 1


def _compiler_params():
  cp = pltpu.CompilerParams()
  if "needs_layout_passes" in pltpu.CompilerParams.__dataclass_fields__:
    cp = dataclasses.replace(cp, needs_layout_passes=False)
  return cp


def _scalar_lane(vec16, lane):
  return jnp.sum(jnp.where(jax.lax.iota(jnp.int32, 16) == lane, vec16, 0))


def _make_sc_plan(chunk, n_passes, capb):
  mesh = plsc.VectorSubcoreMesh(core_axis_name="c", subcore_axis_name="s")
  NV = TILE_E // 16

  scratch = [
      pltpu.VMEM((TILE_E,), jnp.int32),
      pltpu.VMEM((TILE_E,), jnp.int32),
      pltpu.VMEM((capb, BLK), jnp.int32),
      pltpu.VMEM((capb, BLK), jnp.int32),
      pltpu.VMEM((16,), jnp.int32),
      pltpu.VMEM((16,), jnp.int32),
  ]

  out_type = (jax.ShapeDtypeStruct((NW, capb, BLK), jnp.int32),
              jax.ShapeDtypeStruct((NW, capb, BLK), jnp.int32),
              jax.ShapeDtypeStruct((NW, 16), jnp.int32))

  @functools.partial(pl.kernel, out_type=out_type, mesh=mesh,
                     scratch_types=scratch, compiler_params=_compiler_params())
  def plan(src_hbm, dst_hbm, csrc_out, cloc_out, bs_out,
           src_v, dst_v, csrc, cloc, cur_v, bs_v):
    c = jax.lax.axis_index("c")
    s = jax.lax.axis_index("s")
    w = c * 16 + s
    base = w * TILE_E

    pltpu.sync_copy(src_hbm.at[pl.ds(base, TILE_E)], src_v)
    pltpu.sync_copy(dst_hbm.at[pl.ds(base, TILE_E)], dst_v)
    lanes = jax.lax.iota(jnp.int32, 16)
    cur_v[...] = jnp.zeros((16,), jnp.int32)
    bs_v[...] = jnp.zeros((16,), jnp.int32)

    for p in range(n_passes):
      lo = p * chunk
      hi = lo + chunk

      @pl.loop(0, NV)
      def _(v):
        d16 = dst_v[pl.ds(v * 16, 16)]
        m = (d16 >= lo) & (d16 < hi)
        pc = plsc.all_reduce_population_count(m)
        off16 = cur_v[...]
        pos = off16 + plsc.cumsum(m.astype(jnp.int32)) - 1
        row = jax.lax.shift_right_logical(pos, 7)
        col = jax.lax.bitwise_and(pos, BLK - 1)
        s16 = src_v[pl.ds(v * 16, 16)]
        plsc.store_scatter(csrc, [row, col], s16, mask=m)
        plsc.store_scatter(cloc, [row, col], d16 - lo, mask=m)
        cur_v[...] = off16 + pc

      off = _scalar_lane(cur_v[...], 0)
      for k in range(BLK // 16):
        pos = off + k * 16 + lanes
        row = jax.lax.shift_right_logical(pos, 7)
        col = jax.lax.bitwise_and(pos, BLK - 1)
        plsc.store_scatter(csrc, [row, col], jnp.zeros((16,), jnp.int32))
        plsc.store_scatter(cloc, [row, col],
                           jnp.full((16,), chunk, jnp.int32))
      nxt = jax.lax.shift_left(
          jax.lax.shift_right_logical(cur_v[...] + (BLK - 1), 7), 7)
      cur_v[...] = nxt
      bs_v[...] = jnp.where(lanes == p + 1,
                            jax.lax.shift_right_logical(nxt, 7), bs_v[...])

    pltpu.sync_copy(csrc, csrc_out.at[w])
    pltpu.sync_copy(cloc, cloc_out.at[w])
    pltpu.sync_copy(bs_v, bs_out.at[w])

  return plan


def _make_sc_aggr(n_pad, chunk, n_passes, capb, gather=True):
  mesh = plsc.VectorSubcoreMesh(core_axis_name="c", subcore_axis_name="s")
  acc_rows = chunk + BLK
  zrows = acc_rows // 16
  srows = chunk // 16

  scratch = [
      pltpu.VMEM((capb, BLK), jnp.int32) if gather else None,
      pltpu.VMEM((capb, BLK), jnp.int32),
      pltpu.VMEM((16,), jnp.int32),
      pltpu.VMEM((BLK, D), jnp.float32),
      pltpu.VMEM((BLK, D), jnp.float32) if gather else None,
      pltpu.VMEM_SHARED((acc_rows, D), jnp.float32),
      pltpu.SemaphoreType.DMA,
      pltpu.SemaphoreType.DMA,
      pltpu.SemaphoreType.DMA,
      pltpu.SemaphoreType.DMA,
      pltpu.SemaphoreType.DMA,
  ]
  scratch = [t for t in scratch if t is not None]

  @functools.partial(pl.kernel,
                     out_type=jax.ShapeDtypeStruct((2, n_pad, D), jnp.float32),
                     mesh=mesh, scratch_types=scratch,
                     compiler_params=_compiler_params())
  def aggr(x_hbm, csrc_hbm, cloc_hbm, bs_hbm, zeros_hbm, s_out, *rest):
    if gather:
      (csrc, cloc, bs_v, rows0, rows1, acc, sg0, sg1, ss0, ss1, sw) = rest
    else:
      (cloc, bs_v, rows0, acc, sg0, sg1, ss0, ss1, sw) = rest
      csrc = cloc
      rows1 = rows0
    c = jax.lax.axis_index("c")
    s = jax.lax.axis_index("s")
    w = c * 16 + s

    if gather:
      pltpu.sync_copy(csrc_hbm.at[w], csrc)
    pltpu.sync_copy(cloc_hbm.at[w], cloc)
    pltpu.sync_copy(bs_hbm.at[w], bs_v)

    def g_start(b, buf, sem):
      pltpu.async_copy(x_hbm.at[csrc.at[b]], buf, sem)

    def g_wait(b, buf, sem):
      pltpu.make_async_copy(x_hbm.at[csrc.at[b]], buf, sem).wait()

    def s_start(b, buf, sem):
      pltpu.async_copy(buf, acc.at[cloc.at[b]], sem, add=True)

    def s_wait(b, buf, sem):
      pltpu.make_async_copy(buf, acc.at[cloc.at[b]], sem).wait()

    for p in range(n_passes):
      lo = p * chunk

      pltpu.sync_copy(zeros_hbm, rows0)
      zfull, zrem = divmod(zrows, BLK)
      for k in range(zfull):
        pltpu.async_copy(rows0, acc.at[pl.ds(s * zrows + k * BLK, BLK)], sw)
      if zrem:
        pltpu.async_copy(rows0.at[pl.ds(0, zrem)],
                         acc.at[pl.ds(s * zrows + zfull * BLK, zrem)], sw)
      for k in range(zfull):
        pltpu.make_async_copy(rows0,
                              acc.at[pl.ds(s * zrows + k * BLK, BLK)],
                              sw).wait()
      if zrem:
        pltpu.make_async_copy(rows0.at[pl.ds(0, zrem)],
                              acc.at[pl.ds(s * zrows + zfull * BLK, zrem)],
                              sw).wait()
      if not gather:
        pltpu.sync_copy(x_hbm, rows0)
      b0 = _scalar_lane(bs_v[...], p)
      nk = _scalar_lane(bs_v[...], p + 1) - b0
      plsc.subcore_barrier()

      if gather:
        @pl.when(nk > 0)
        def _():
          g_start(b0, rows0, sg0)

        @pl.loop(0, NBLK)
        def _(k):
          @pl.when(k < nk)
          def _():
            b = b0 + k
            even = jax.lax.bitwise_and(k, 1) == 0
            odd = jnp.logical_not(even)

            @pl.when((k >= 1) & even)
            def _():
              s_wait(b - 1, rows1, ss1)

            @pl.when((k >= 1) & odd)
            def _():
              s_wait(b - 1, rows0, ss0)

            @pl.when((k + 1 < nk) & even)
            def _():
              g_start(b + 1, rows1, sg1)

            @pl.when((k + 1 < nk) & odd)
            def _():
              g_start(b + 1, rows0, sg0)

            @pl.when(even)
            def _():
              g_wait(b, rows0, sg0)
              s_start(b, rows0, ss0)

            @pl.when(odd)
            def _():
              g_wait(b, rows1, sg1)
              s_start(b, rows1, ss1)

        @pl.when(nk > 0)
        def _():
          last = nk - 1
          le = jax.lax.bitwise_and(last, 1) == 0

          @pl.when(le)
          def _():
            s_wait(b0 + last, rows0, ss0)

          @pl.when(jnp.logical_not(le))
          def _():
            s_wait(b0 + last, rows1, ss1)
      else:
        @pl.loop(0, NBLK)
        def _(k):
          @pl.when(k < nk)
          def _():
            @pl.when(k >= 1)
            def _():
              s_wait(b0 + k - 1, rows0, ss0)
            s_start(b0 + k, rows0, ss0)

        @pl.when(nk > 0)
        def _():
          s_wait(b0 + nk - 1, rows0, ss0)

      plsc.subcore_barrier()

      wfull, wrem = divmod(srows, BLK)
      for k in range(wfull):
        r0 = s * srows + k * BLK
        pltpu.async_copy(acc.at[pl.ds(r0, BLK)],
                         s_out.at[c, pl.ds(lo + r0, BLK)], sw)
      if wrem:
        r0 = s * srows + wfull * BLK
        pltpu.async_copy(acc.at[pl.ds(r0, wrem)],
                         s_out.at[c, pl.ds(lo + r0, wrem)], sw)
      for k in range(wfull):
        r0 = s * srows + k * BLK
        pltpu.make_async_copy(acc.at[pl.ds(r0, BLK)],
                              s_out.at[c, pl.ds(lo + r0, BLK)], sw).wait()
      if wrem:
        r0 = s * srows + wfull * BLK
        pltpu.make_async_copy(acc.at[pl.ds(r0, wrem)],
                              s_out.at[c, pl.ds(lo + r0, wrem)], sw).wait()
      plsc.subcore_barrier()

  return aggr


def _tc_layer(s0, s1, c0, c1, x, w_l, w_r, b, relu, norm, blk=512):
  n = s0.shape[0]

  def body(s0_r, s1_r, c0_r, c1_r, x_r, wl_r, wr_r, b_r, o_r):
    cnt = c0_r[:, 0:1] + c1_r[:, 0:1]
    a = (s0_r[...] + s1_r[...]) / jnp.maximum(cnt, 1.0)
    h = (jnp.dot(a, wl_r[...], preferred_element_type=jnp.float32)
         + jnp.dot(x_r[...], wr_r[...], preferred_element_type=jnp.float32)
         + b_r[...])
    if relu:
      h = jnp.maximum(h, 0.0)
    if norm:
      nrm = jnp.sqrt(jnp.sum(h * h, axis=-1, keepdims=True))
      h = h / jnp.maximum(nrm, 1e-12)
    o_r[...] = h

  return pl.pallas_call(
      body,
      grid=(n // blk,),
      in_specs=[
          pl.BlockSpec((blk, D), lambda i: (i, 0)),
          pl.BlockSpec((blk, D), lambda i: (i, 0)),
          pl.BlockSpec((blk, D), lambda i: (i, 0)),
          pl.BlockSpec((blk, D), lambda i: (i, 0)),
          pl.BlockSpec((blk, D), lambda i: (i, 0)),
          pl.BlockSpec((D, D), lambda i: (0, 0)),
          pl.BlockSpec((D, D), lambda i: (0, 0)),
          pl.BlockSpec((1, D), lambda i: (0, 0)),
      ],
      out_specs=pl.BlockSpec((blk, D), lambda i: (i, 0)),
      out_shape=jax.ShapeDtypeStruct((n, D), jnp.float32),
  )(s0, s1, c0, c1, x, w_l, w_r, b)


def kernel(x_drug, x_protein, edge_index_dp, edge_index_pd,
           W1_dp_l, W1_dp_r, b1_dp, W1_pd_l, W1_pd_r, b1_pd,
           W2_dp_l, W2_dp_r, b2_dp, W2_pd_l, W2_pd_r, b2_pd):
  f32 = jnp.float32
  i32 = jnp.int32

  def pad_edges(ei, dst_pad_val):
    src = jnp.pad(ei[0].astype(i32), (0, EPAD - E))
    dst = jnp.pad(ei[1].astype(i32), (0, EPAD - E),
                  constant_values=dst_pad_val)
    return src, dst

  src_dp, dst_dp = pad_edges(edge_index_dp, P_PAD)
  src_pd, dst_pd = pad_edges(edge_index_pd, D_PAD)

  x_p = jnp.pad(x_protein.astype(f32), ((0, P_PAD - N_PROT), (0, 0)))
  x_d = jnp.pad(x_drug.astype(f32), ((0, D_PAD - N_DRUG), (0, 0)))

  ones = jnp.ones((BLK, D), f32)
  zer = jnp.zeros((BLK, D), f32)

  plan_p = _make_sc_plan(P_CHUNK, P_PASSES, P_CAPB)
  plan_d = _make_sc_plan(D_CHUNK, D_PASSES, D_CAPB)
  aggr_p = _make_sc_aggr(P_PAD, P_CHUNK, P_PASSES, P_CAPB)
  aggr_d = _make_sc_aggr(D_PAD, D_CHUNK, D_PASSES, D_CAPB)
  count_p = _make_sc_aggr(P_PAD, P_CHUNK, P_PASSES, P_CAPB, gather=False)
  count_d = _make_sc_aggr(D_PAD, D_CHUNK, D_PASSES, D_CAPB, gather=False)

  csrc_p, cloc_p, bs_p = plan_p(src_dp, dst_dp)
  csrc_d, cloc_d, bs_d = plan_d(src_pd, dst_pd)

  c_p = count_p(ones, csrc_p, cloc_p, bs_p, zer)
  c_d = count_d(ones, csrc_d, cloc_d, bs_d, zer)

  s1p = aggr_p(x_drug, csrc_p, cloc_p, bs_p, zer)
  s1d = aggr_d(x_protein, csrc_d, cloc_d, bs_d, zer)
  h_p = _tc_layer(s1p[0], s1p[1], c_p[0], c_p[1], x_p,
                  W1_dp_l, W1_dp_r, b1_dp.reshape(1, D), True, False)
  h_d = _tc_layer(s1d[0], s1d[1], c_d[0], c_d[1], x_d,
                  W1_pd_l, W1_pd_r, b1_pd.reshape(1, D), True, False)

  s2p = aggr_p(h_d, csrc_p, cloc_p, bs_p, zer)
  s2d = aggr_d(h_p, csrc_d, cloc_d, bs_d, zer)
  z_p = _tc_layer(s2p[0], s2p[1], c_p[0], c_p[1], h_p,
                  W2_dp_l, W2_dp_r, b2_dp.reshape(1, D), False, True)
  z_d = _tc_layer(s2d[0], s2d[1], c_d[0], c_d[1], h_d,
                  W2_pd_l, W2_pd_r, b2_pd.reshape(1, D), False, True)

  return (z_d[:N_DRUG], z_p[:N_PROT])
